# baseline (speedup 1.0000x reference)
import jax
import jax.numpy as jnp
from jax.experimental import pallas as pl
from jax.experimental.pallas import tpu as pltpu


def _main_body(adj_ref, sadj_ref, x_full_ref, x_blk_ref, wmlp_ref,
               wA_ref, wAs_ref, outA_ref, outAs_ref, colsum_ref,
               xwA_s, xwAs_s):
    i = pl.program_id(0)

    @pl.when(i == 0)
    def _precompute():
        xf = x_full_ref[...]
        xwA_s[...] = jnp.dot(
            xf, wA_ref[...], preferred_element_type=jnp.float32)
        xwAs_s[...] = jnp.dot(
            xf, wAs_ref[...], preferred_element_type=jnp.float32)

    a = jnp.maximum(
        jnp.dot(adj_ref[...], xwA_s[...],
                preferred_element_type=jnp.float32), 0.0)
    b = jnp.maximum(
        jnp.dot(sadj_ref[...], xwAs_s[...],
                preferred_element_type=jnp.float32), 0.0)
    outA_ref[...] = a.astype(jnp.bfloat16)
    outAs_ref[...] = b.astype(jnp.bfloat16)
    mlp = jnp.maximum(
        jnp.dot(x_blk_ref[...], wmlp_ref[...],
                preferred_element_type=jnp.float32), 0.0)
    part = jnp.sum(a + b + mlp, axis=0, keepdims=True)

    @pl.when(i == 0)
    def _set():
        colsum_ref[0:1, :] = part

    @pl.when(i > 0)
    def _add():
        colsum_ref[0:1, :] += part


def _attn_body(n_total, outA_ref, outAs_ref, x_blk_ref, wmlp_ref,
               colsum_ref, attk_ref, attv_ref, emb_ref):
    tao = 3.0
    d = attk_ref.shape[0]
    kvec = jnp.dot(colsum_ref[0:1, :] * (1.0 / n_total), attk_ref[...],
                   preferred_element_type=jnp.float32)
    kq = jnp.broadcast_to(kvec, (d, d))
    mlp = jnp.maximum(
        jnp.dot(x_blk_ref[...], wmlp_ref[...],
                preferred_element_type=jnp.float32), 0.0)
    oA = outA_ref[...].astype(jnp.float32)
    oAs = outAs_ref[...].astype(jnp.float32)
    nt = (((1,), (1,)), ((), ()))
    s0 = jax.lax.dot_general(mlp, kq, nt,
                             preferred_element_type=jnp.float32)
    s1 = jax.lax.dot_general(oA, kq, nt,
                             preferred_element_type=jnp.float32)
    s2 = jax.lax.dot_general(oAs, kq, nt,
                             preferred_element_type=jnp.float32)
    g0 = jax.nn.sigmoid(s0)
    g1 = jax.nn.sigmoid(s1)
    g2 = jax.nn.sigmoid(s2)
    v = attv_ref
    u1 = (g0 * (v[0:1, 1:2] - v[0:1, 0:1]) +
          g1 * (v[1:2, 1:2] - v[1:2, 0:1]) +
          g2 * (v[2:3, 1:2] - v[2:3, 0:1])) * (1.0 / tao)
    u2 = (g0 * (v[0:1, 2:3] - v[0:1, 0:1]) +
          g1 * (v[1:2, 2:3] - v[1:2, 0:1]) +
          g2 * (v[2:3, 2:3] - v[2:3, 0:1])) * (1.0 / tao)
    r1 = jnp.exp(u1)
    r2 = jnp.exp(u2)
    a0 = 1.0 / (1.0 + r1 + r2)
    emb_ref[...] = a0 * (mlp + r1 * oA + r2 * oAs)


def kernel(inputx, adj, sadj, weight_mlp, weight_A, weight_As,
           att_vec_k, att_vec_v):
    n, d = inputx.shape

    bm = 200 if n % 200 == 0 else n
    ni = n // bm

    outA, outAs, colsum = pl.pallas_call(
        _main_body,
        grid=(ni,),
        in_specs=[
            pl.BlockSpec((bm, n), lambda i: (i, 0)),
            pl.BlockSpec((bm, n), lambda i: (i, 0)),
            pl.BlockSpec((n, d), lambda i: (0, 0)),
            pl.BlockSpec((bm, d), lambda i: (i, 0)),
            pl.BlockSpec((d, d), lambda i: (0, 0)),
            pl.BlockSpec((d, d), lambda i: (0, 0)),
            pl.BlockSpec((d, d), lambda i: (0, 0)),
        ],
        out_specs=[
            pl.BlockSpec((bm, d), lambda i: (i, 0)),
            pl.BlockSpec((bm, d), lambda i: (i, 0)),
            pl.BlockSpec((8, d), lambda i: (0, 0)),
        ],
        out_shape=[
            jax.ShapeDtypeStruct((n, d), jnp.bfloat16),
            jax.ShapeDtypeStruct((n, d), jnp.bfloat16),
            jax.ShapeDtypeStruct((8, d), jnp.float32),
        ],
        scratch_shapes=[
            pltpu.VMEM((n, d), jnp.float32),
            pltpu.VMEM((n, d), jnp.float32),
        ],
        compiler_params=pltpu.CompilerParams(
            vmem_limit_bytes=63 * 1024 * 1024),
    )(adj, sadj, inputx, inputx, weight_mlp, weight_A, weight_As)

    attv_pad = jnp.zeros((8, 128), jnp.float32).at[:3, :3].set(att_vec_v)

    bm2 = n // 5 if n % 5 == 0 else n
    emb = pl.pallas_call(
        lambda *refs: _attn_body(float(n), *refs),
        grid=(n // bm2,),
        in_specs=[
            pl.BlockSpec((bm2, d), lambda i: (i, 0)),
            pl.BlockSpec((bm2, d), lambda i: (i, 0)),
            pl.BlockSpec((bm2, d), lambda i: (i, 0)),
            pl.BlockSpec((d, d), lambda i: (0, 0)),
            pl.BlockSpec((8, d), lambda i: (0, 0)),
            pl.BlockSpec((d, d), lambda i: (0, 0)),
            pl.BlockSpec((8, 128), lambda i: (0, 0)),
        ],
        out_specs=pl.BlockSpec((bm2, d), lambda i: (i, 0)),
        out_shape=jax.ShapeDtypeStruct((n, d), jnp.float32),
    )(outA, outAs, inputx, weight_mlp, colsum, att_vec_k, attv_pad)

    return emb


# speedup vs baseline: 1.0032x; 1.0032x over previous
import jax
import jax.numpy as jnp
from jax.experimental import pallas as pl
from jax.experimental.pallas import tpu as pltpu


def _main_body(adj_ref, sadj_ref, x_full_ref, x_blk_ref, wmlp_ref,
               wA_ref, wAs_ref, outA_ref, outAs_ref, mlp_ref, colsum_ref,
               xwA_s, xwAs_s):
    i = pl.program_id(0)

    @pl.when(i == 0)
    def _precompute():
        xf = x_full_ref[...]
        xwA_s[...] = jnp.dot(
            xf, wA_ref[...], preferred_element_type=jnp.float32)
        xwAs_s[...] = jnp.dot(
            xf, wAs_ref[...], preferred_element_type=jnp.float32)

    a = jnp.maximum(
        jnp.dot(adj_ref[...], xwA_s[...],
                preferred_element_type=jnp.float32), 0.0)
    b = jnp.maximum(
        jnp.dot(sadj_ref[...], xwAs_s[...],
                preferred_element_type=jnp.float32), 0.0)
    outA_ref[...] = a.astype(jnp.bfloat16)
    outAs_ref[...] = b.astype(jnp.bfloat16)
    mlp = jnp.maximum(
        jnp.dot(x_blk_ref[...], wmlp_ref[...],
                preferred_element_type=jnp.float32), 0.0)
    mlp_ref[...] = mlp.astype(jnp.bfloat16)
    part = jnp.sum(a + b + mlp, axis=0, keepdims=True)

    @pl.when(i == 0)
    def _set():
        colsum_ref[0:1, :] = part

    @pl.when(i > 0)
    def _add():
        colsum_ref[0:1, :] += part


def _attn_body(n_total, outA_ref, outAs_ref, mlp_ref,
               colsum_ref, attk_ref, attv_ref, emb_ref):
    tao = 3.0
    d = attk_ref.shape[0]
    kvec = jnp.dot(colsum_ref[0:1, :] * (1.0 / n_total), attk_ref[...],
                   preferred_element_type=jnp.float32)
    kq = jnp.broadcast_to(kvec, (d, d))
    mlp = mlp_ref[...].astype(jnp.float32)
    oA = outA_ref[...].astype(jnp.float32)
    oAs = outAs_ref[...].astype(jnp.float32)
    nt = (((1,), (1,)), ((), ()))
    s0 = jax.lax.dot_general(mlp, kq, nt,
                             preferred_element_type=jnp.float32)
    s1 = jax.lax.dot_general(oA, kq, nt,
                             preferred_element_type=jnp.float32)
    s2 = jax.lax.dot_general(oAs, kq, nt,
                             preferred_element_type=jnp.float32)
    g0 = jax.nn.sigmoid(s0)
    g1 = jax.nn.sigmoid(s1)
    g2 = jax.nn.sigmoid(s2)
    v = attv_ref
    u1 = (g0 * (v[0:1, 1:2] - v[0:1, 0:1]) +
          g1 * (v[1:2, 1:2] - v[1:2, 0:1]) +
          g2 * (v[2:3, 1:2] - v[2:3, 0:1])) * (1.0 / tao)
    u2 = (g0 * (v[0:1, 2:3] - v[0:1, 0:1]) +
          g1 * (v[1:2, 2:3] - v[1:2, 0:1]) +
          g2 * (v[2:3, 2:3] - v[2:3, 0:1])) * (1.0 / tao)
    r1 = jnp.exp(u1)
    r2 = jnp.exp(u2)
    a0 = 1.0 / (1.0 + r1 + r2)
    emb_ref[...] = a0 * (mlp + r1 * oA + r2 * oAs)


def kernel(inputx, adj, sadj, weight_mlp, weight_A, weight_As,
           att_vec_k, att_vec_v):
    n, d = inputx.shape

    bm = 200 if n % 200 == 0 else n
    ni = n // bm

    outA, outAs, mlp, colsum = pl.pallas_call(
        _main_body,
        grid=(ni,),
        in_specs=[
            pl.BlockSpec((bm, n), lambda i: (i, 0)),
            pl.BlockSpec((bm, n), lambda i: (i, 0)),
            pl.BlockSpec((n, d), lambda i: (0, 0)),
            pl.BlockSpec((bm, d), lambda i: (i, 0)),
            pl.BlockSpec((d, d), lambda i: (0, 0)),
            pl.BlockSpec((d, d), lambda i: (0, 0)),
            pl.BlockSpec((d, d), lambda i: (0, 0)),
        ],
        out_specs=[
            pl.BlockSpec((bm, d), lambda i: (i, 0)),
            pl.BlockSpec((bm, d), lambda i: (i, 0)),
            pl.BlockSpec((bm, d), lambda i: (i, 0)),
            pl.BlockSpec((8, d), lambda i: (0, 0)),
        ],
        out_shape=[
            jax.ShapeDtypeStruct((n, d), jnp.bfloat16),
            jax.ShapeDtypeStruct((n, d), jnp.bfloat16),
            jax.ShapeDtypeStruct((n, d), jnp.bfloat16),
            jax.ShapeDtypeStruct((8, d), jnp.float32),
        ],
        scratch_shapes=[
            pltpu.VMEM((n, d), jnp.float32),
            pltpu.VMEM((n, d), jnp.float32),
        ],
        compiler_params=pltpu.CompilerParams(
            vmem_limit_bytes=63 * 1024 * 1024),
    )(adj, sadj, inputx, inputx, weight_mlp, weight_A, weight_As)

    attv_pad = jnp.zeros((8, 128), jnp.float32).at[:3, :3].set(att_vec_v)

    bm2 = n // 5 if n % 5 == 0 else n
    emb = pl.pallas_call(
        lambda *refs: _attn_body(float(n), *refs),
        grid=(n // bm2,),
        in_specs=[
            pl.BlockSpec((bm2, d), lambda i: (i, 0)),
            pl.BlockSpec((bm2, d), lambda i: (i, 0)),
            pl.BlockSpec((bm2, d), lambda i: (i, 0)),
            pl.BlockSpec((8, d), lambda i: (0, 0)),
            pl.BlockSpec((d, d), lambda i: (0, 0)),
            pl.BlockSpec((8, 128), lambda i: (0, 0)),
        ],
        out_specs=pl.BlockSpec((bm2, d), lambda i: (i, 0)),
        out_shape=jax.ShapeDtypeStruct((n, d), jnp.float32),
    )(outA, outAs, mlp, colsum, att_vec_k, attv_pad)

    return emb


# speedup vs baseline: 1.0076x; 1.0044x over previous
import jax
import jax.numpy as jnp
from jax.experimental import pallas as pl
from jax.experimental.pallas import tpu as pltpu


def _main_body(bm, adj_ref, sadj_ref, x_full_ref, wmlp_ref,
               wA_ref, wAs_ref, outA_ref, outAs_ref, mlp_ref, colsum_ref,
               xwA_s, xwAs_s):
    i = pl.program_id(0)

    @pl.when(i == 0)
    def _precompute():
        xf = x_full_ref[...]
        xwA_s[...] = jnp.dot(
            xf, wA_ref[...], preferred_element_type=jnp.float32)
        xwAs_s[...] = jnp.dot(
            xf, wAs_ref[...], preferred_element_type=jnp.float32)

    a = jnp.maximum(
        jnp.dot(adj_ref[...], xwA_s[...],
                preferred_element_type=jnp.float32), 0.0)
    b = jnp.maximum(
        jnp.dot(sadj_ref[...], xwAs_s[...],
                preferred_element_type=jnp.float32), 0.0)
    outA_ref[...] = a.astype(jnp.bfloat16)
    outAs_ref[...] = b.astype(jnp.bfloat16)
    mlp = jnp.maximum(
        jnp.dot(x_full_ref[pl.ds(i * bm, bm), :], wmlp_ref[...],
                preferred_element_type=jnp.float32), 0.0)
    mlp_ref[...] = mlp.astype(jnp.bfloat16)
    part = jnp.sum(a + b + mlp, axis=0, keepdims=True)

    @pl.when(i == 0)
    def _set():
        colsum_ref[0:1, :] = part

    @pl.when(i > 0)
    def _add():
        colsum_ref[0:1, :] += part


def _attn_body(n_total, outA_ref, outAs_ref, mlp_ref,
               colsum_ref, attk_ref, attv_ref, emb_ref):
    d = attk_ref.shape[0]
    kvec = jnp.dot(colsum_ref[0:1, :] * (1.0 / n_total), attk_ref[...],
                   preferred_element_type=jnp.float32)
    kq = jnp.broadcast_to(kvec, (d, d))
    mlp = mlp_ref[...].astype(jnp.float32)
    oA = outA_ref[...].astype(jnp.float32)
    oAs = outAs_ref[...].astype(jnp.float32)
    nt = (((1,), (1,)), ((), ()))
    s0 = jax.lax.dot_general(mlp, kq, nt,
                             preferred_element_type=jnp.float32)
    s1 = jax.lax.dot_general(oA, kq, nt,
                             preferred_element_type=jnp.float32)
    s2 = jax.lax.dot_general(oAs, kq, nt,
                             preferred_element_type=jnp.float32)
    g0 = jax.nn.sigmoid(s0)
    g1 = jax.nn.sigmoid(s1)
    g2 = jax.nn.sigmoid(s2)
    v = attv_ref
    u1 = g0 * v[0:1, 1:2] + g1 * v[1:2, 1:2] + g2 * v[2:3, 1:2]
    u2 = g0 * v[0:1, 2:3] + g1 * v[1:2, 2:3] + g2 * v[2:3, 2:3]
    r1 = jnp.exp(u1)
    r2 = jnp.exp(u2)
    a0 = 1.0 / (1.0 + r1 + r2)
    emb_ref[...] = a0 * (mlp + r1 * oA + r2 * oAs)


def kernel(inputx, adj, sadj, weight_mlp, weight_A, weight_As,
           att_vec_k, att_vec_v):
    n, d = inputx.shape

    bm = 200 if n % 200 == 0 else n
    ni = n // bm

    outA, outAs, mlp, colsum = pl.pallas_call(
        lambda *refs: _main_body(bm, *refs),
        grid=(ni,),
        in_specs=[
            pl.BlockSpec((bm, n), lambda i: (i, 0)),
            pl.BlockSpec((bm, n), lambda i: (i, 0)),
            pl.BlockSpec((n, d), lambda i: (0, 0)),
            pl.BlockSpec((d, d), lambda i: (0, 0)),
            pl.BlockSpec((d, d), lambda i: (0, 0)),
            pl.BlockSpec((d, d), lambda i: (0, 0)),
        ],
        out_specs=[
            pl.BlockSpec((bm, d), lambda i: (i, 0)),
            pl.BlockSpec((bm, d), lambda i: (i, 0)),
            pl.BlockSpec((bm, d), lambda i: (i, 0)),
            pl.BlockSpec((8, d), lambda i: (0, 0)),
        ],
        out_shape=[
            jax.ShapeDtypeStruct((n, d), jnp.bfloat16),
            jax.ShapeDtypeStruct((n, d), jnp.bfloat16),
            jax.ShapeDtypeStruct((n, d), jnp.bfloat16),
            jax.ShapeDtypeStruct((8, d), jnp.float32),
        ],
        scratch_shapes=[
            pltpu.VMEM((n, d), jnp.float32),
            pltpu.VMEM((n, d), jnp.float32),
        ],
        compiler_params=pltpu.CompilerParams(
            vmem_limit_bytes=63 * 1024 * 1024),
    )(adj, sadj, inputx, weight_mlp, weight_A, weight_As)

    tao = 3.0
    attv_delta = (att_vec_v - att_vec_v[:, 0:1]) / tao
    attv_pad = jnp.zeros((8, 128), jnp.float32).at[:3, :3].set(attv_delta)

    bm2 = n // 5 if n % 5 == 0 else n
    emb = pl.pallas_call(
        lambda *refs: _attn_body(float(n), *refs),
        grid=(n // bm2,),
        in_specs=[
            pl.BlockSpec((bm2, d), lambda i: (i, 0)),
            pl.BlockSpec((bm2, d), lambda i: (i, 0)),
            pl.BlockSpec((bm2, d), lambda i: (i, 0)),
            pl.BlockSpec((8, d), lambda i: (0, 0)),
            pl.BlockSpec((d, d), lambda i: (0, 0)),
            pl.BlockSpec((8, 128), lambda i: (0, 0)),
        ],
        out_specs=pl.BlockSpec((bm2, d), lambda i: (i, 0)),
        out_shape=jax.ShapeDtypeStruct((n, d), jnp.float32),
    )(outA, outAs, mlp, colsum, att_vec_k, attv_pad)

    return emb


# speedup vs baseline: 1.0198x; 1.0120x over previous
import jax
import jax.numpy as jnp
from jax.experimental import pallas as pl
from jax.experimental.pallas import tpu as pltpu


def _body(bm, ni, bm2, n_total,
          adj_ref, sadj_ref, x_full_ref, wmlp_ref, wA_ref, wAs_ref,
          attk_ref, attv_ref,
          emb_ref, colsum_ref,
          xwA_s, xwAs_s, outA_s, outAs_s, mlp_s):
    i = pl.program_id(0)

    @pl.when(i == 0)
    def _precompute():
        xf = x_full_ref[...]
        xwA_s[...] = jnp.dot(
            xf, wA_ref[...], preferred_element_type=jnp.float32)
        xwAs_s[...] = jnp.dot(
            xf, wAs_ref[...], preferred_element_type=jnp.float32)

    @pl.when(i < ni)
    def _phase1():
        a = jnp.maximum(
            jnp.dot(adj_ref[...], xwA_s[...],
                    preferred_element_type=jnp.float32), 0.0)
        b = jnp.maximum(
            jnp.dot(sadj_ref[...], xwAs_s[...],
                    preferred_element_type=jnp.float32), 0.0)
        mlp = jnp.maximum(
            jnp.dot(x_full_ref[pl.ds(i * bm, bm), :], wmlp_ref[...],
                    preferred_element_type=jnp.float32), 0.0)
        sl = pl.ds(i * bm, bm)
        outA_s[sl, :] = a.astype(jnp.bfloat16)
        outAs_s[sl, :] = b.astype(jnp.bfloat16)
        mlp_s[sl, :] = mlp.astype(jnp.bfloat16)
        part = jnp.sum(a + b + mlp, axis=0, keepdims=True)

        @pl.when(i == 0)
        def _set():
            colsum_ref[0:1, :] = part

        @pl.when(i > 0)
        def _add():
            colsum_ref[0:1, :] += part

    @pl.when(i >= ni)
    def _phase2():
        d = attk_ref.shape[0]
        sl = pl.ds((i - ni) * bm2, bm2)
        kvec = jnp.dot(colsum_ref[0:1, :] * (1.0 / n_total), attk_ref[...],
                       preferred_element_type=jnp.float32)
        kq = jnp.broadcast_to(kvec, (d, d))
        mlp = mlp_s[sl, :].astype(jnp.float32)
        oA = outA_s[sl, :].astype(jnp.float32)
        oAs = outAs_s[sl, :].astype(jnp.float32)
        nt = (((1,), (1,)), ((), ()))
        s0 = jax.lax.dot_general(mlp, kq, nt,
                                 preferred_element_type=jnp.float32)
        s1 = jax.lax.dot_general(oA, kq, nt,
                                 preferred_element_type=jnp.float32)
        s2 = jax.lax.dot_general(oAs, kq, nt,
                                 preferred_element_type=jnp.float32)
        g0 = jax.nn.sigmoid(s0)
        g1 = jax.nn.sigmoid(s1)
        g2 = jax.nn.sigmoid(s2)
        v = attv_ref
        u1 = g0 * v[0:1, 1:2] + g1 * v[1:2, 1:2] + g2 * v[2:3, 1:2]
        u2 = g0 * v[0:1, 2:3] + g1 * v[1:2, 2:3] + g2 * v[2:3, 2:3]
        r1 = jnp.exp(u1)
        r2 = jnp.exp(u2)
        a0 = 1.0 / (1.0 + r1 + r2)
        emb_ref[...] = a0 * (mlp + r1 * oA + r2 * oAs)


def kernel(inputx, adj, sadj, weight_mlp, weight_A, weight_As,
           att_vec_k, att_vec_v):
    n, d = inputx.shape

    bm = 200 if n % 200 == 0 else n
    ni = n // bm
    bm2 = n // 5 if n % 5 == 0 else n
    nf = n // bm2

    tao = 3.0
    attv_delta = (att_vec_v - att_vec_v[:, 0:1]) / tao
    attv_pad = jnp.zeros((8, 128), jnp.float32).at[:3, :3].set(attv_delta)

    def body(*refs):
        return _body(bm, ni, bm2, float(n), *refs)

    emb, _colsum = pl.pallas_call(
        body,
        grid=(ni + nf,),
        in_specs=[
            pl.BlockSpec((bm, n), lambda i: (jnp.minimum(i, ni - 1), 0)),
            pl.BlockSpec((bm, n), lambda i: (jnp.minimum(i, ni - 1), 0)),
            pl.BlockSpec((n, d), lambda i: (0, 0)),
            pl.BlockSpec((d, d), lambda i: (0, 0)),
            pl.BlockSpec((d, d), lambda i: (0, 0)),
            pl.BlockSpec((d, d), lambda i: (0, 0)),
            pl.BlockSpec((d, d), lambda i: (0, 0)),
            pl.BlockSpec((8, 128), lambda i: (0, 0)),
        ],
        out_specs=[
            pl.BlockSpec((bm2, d), lambda i: (jnp.maximum(i - ni, 0), 0)),
            pl.BlockSpec((8, d), lambda i: (0, 0)),
        ],
        out_shape=[
            jax.ShapeDtypeStruct((n, d), jnp.float32),
            jax.ShapeDtypeStruct((8, d), jnp.float32),
        ],
        scratch_shapes=[
            pltpu.VMEM((n, d), jnp.float32),
            pltpu.VMEM((n, d), jnp.float32),
            pltpu.VMEM((n, d), jnp.bfloat16),
            pltpu.VMEM((n, d), jnp.bfloat16),
            pltpu.VMEM((n, d), jnp.bfloat16),
        ],
        compiler_params=pltpu.CompilerParams(
            vmem_limit_bytes=63 * 1024 * 1024),
    )(adj, sadj, inputx, weight_mlp, weight_A, weight_As,
      att_vec_k, attv_pad)

    return emb
